# Initial kernel scaffold; baseline (speedup 1.0000x reference)
import functools
import math

import jax
import jax.numpy as jnp
from jax import lax
from jax.experimental import pallas as pl
from jax.experimental.pallas import tpu as pltpu
from jax.experimental.pallas import tpu_sc as plsc

B, M, D, K = 1024, 16384, 512, 32
RB = 128
KC = 1024
NR = B // RB
NKC = M // KC

NEG = jnp.float32(-jnp.inf)


def _score_topk_body(q_ref, wq_ref, bq_ref, keys_ref, phi_ref,
                     idx_ref, w_ref, proj_ref, scores_ref, vals_ref):
    j = pl.program_id(1)

    @pl.when(j == 0)
    def _():
        p = lax.dot_general(q_ref[...], wq_ref[...], (((1,), (1,)), ((), ())),
                            preferred_element_type=jnp.float32)
        proj_ref[...] = p + bq_ref[...]

    s = lax.dot_general(proj_ref[...], keys_ref[...], (((1,), (1,)), ((), ())),
                        preferred_element_type=jnp.float32)
    scores_ref[j] = s * jnp.float32(0.7 / math.sqrt(D)) + phi_ref[...]

    @pl.when(j == NKC - 1)
    def _():
        jidx = lax.broadcasted_iota(jnp.int32, (NKC, RB, KC), 0)
        lidx = lax.broadcasted_iota(jnp.int32, (NKC, RB, KC), 2)
        gcol = jidx * KC + lidx

        def body(k, carry):
            sc = scores_ref[...]
            m = jnp.max(jnp.max(sc, axis=2), axis=0)
            cand = jnp.where(sc == m[None, :, None], gcol, jnp.int32(M))
            idxk = jnp.min(jnp.min(cand, axis=2), axis=0)
            vals_ref[k] = m
            idx_ref[k] = idxk
            scores_ref[...] = jnp.where(gcol == idxk[None, :, None], NEG, sc)
            return carry

        lax.fori_loop(0, K, body, 0)

        vals = vals_ref[...]
        mx = jnp.max(vals, axis=0)
        e = jnp.exp(vals - mx[None, :])
        w_ref[...] = e / jnp.sum(e, axis=0)[None, :]


_score_topk = pl.pallas_call(
    _score_topk_body,
    grid=(NR, NKC),
    in_specs=[
        pl.BlockSpec((RB, D), lambda i, j: (i, 0)),
        pl.BlockSpec((D, D), lambda i, j: (0, 0)),
        pl.BlockSpec((1, D), lambda i, j: (0, 0)),
        pl.BlockSpec((KC, D), lambda i, j: (j, 0)),
        pl.BlockSpec((1, KC), lambda i, j: (0, j)),
    ],
    out_specs=[
        pl.BlockSpec((K, RB), lambda i, j: (0, i)),
        pl.BlockSpec((K, RB), lambda i, j: (0, i)),
    ],
    out_shape=[
        jax.ShapeDtypeStruct((K, B), jnp.int32),
        jax.ShapeDtypeStruct((K, B), jnp.float32),
    ],
    scratch_shapes=[
        pltpu.VMEM((RB, D), jnp.float32),
        pltpu.VMEM((NKC, RB, KC), jnp.float32),
        pltpu.VMEM((K, RB), jnp.float32),
    ],
    compiler_params=pltpu.CompilerParams(
        dimension_semantics=("parallel", "arbitrary")),
)


SC_CORES, SC_SUBCORES = 2, 16
NW = SC_CORES * SC_SUBCORES
ROWS = B * K
B_PER_W = ROWS // NW
CH = 128
NCHUNK = B_PER_W // CH

_sc_mesh = plsc.VectorSubcoreMesh(core_axis_name="c", subcore_axis_name="s")


@functools.partial(
    pl.kernel,
    mesh=_sc_mesh,
    out_type=jax.ShapeDtypeStruct((ROWS, D), jnp.float32),
    scratch_types=[
        pltpu.VMEM((CH,), jnp.int32),
        pltpu.VMEM((CH, D), jnp.float32),
        pltpu.SemaphoreType.DMA,
    ],
)
def _gather_rows(idx_hbm, table_hbm, out_hbm, idx_v, rows_v, sem):
    wid = lax.axis_index("s") * SC_CORES + lax.axis_index("c")
    base = wid * B_PER_W
    for t in range(NCHUNK):
        off = base + t * CH
        pltpu.sync_copy(idx_hbm.at[pl.ds(off, CH)], idx_v)
        pltpu.async_copy(table_hbm.at[idx_v], rows_v, sem).wait()
        pltpu.sync_copy(rows_v, out_hbm.at[pl.ds(off, CH)])


def kernel(query, Wq, bq, memory_keys, memory_values, phi_scores, top_k):
    del top_k
    phi_b = (phi_scores * jnp.float32(0.3)
             / (jnp.max(phi_scores) + jnp.float32(1e-8))).reshape(1, M)
    idx_t, w_t = _score_topk(query, Wq, bq.reshape(1, D), memory_keys, phi_b)
    idx = idx_t.T.reshape(ROWS)
    retrieved = _gather_rows(idx, memory_values).reshape(B, K, D)
    return retrieved, w_t.T


# speedup vs baseline: 4.6243x; 4.6243x over previous
import functools
import math

import jax
import jax.numpy as jnp
from jax import lax
from jax.experimental import pallas as pl
from jax.experimental.pallas import tpu as pltpu
from jax.experimental.pallas import tpu_sc as plsc

B, M, D, K = 1024, 16384, 512, 32
RB1 = 512
KC = 1024
NR1 = B // RB1
NKC = M // KC
CW = 128
NCH = M // CW
SUB = KC // CW
NCAND = 36
CANDW = NCAND * CW

RB2 = 256
NR2 = B // RB2

NEG = float("-inf")
BIG = 1 << 30


def _score_cand_body(q_ref, wq_ref, bq_ref, keys_ref, phi_ref,
                     s3_ref, cid_ref, gidx_ref, proj_ref, scores_ref):
    i = pl.program_id(0)
    j = pl.program_id(1)

    @pl.when(j == 0)
    def _():
        p = lax.dot_general(q_ref[...], wq_ref[...], (((1,), (1,)), ((), ())),
                            preferred_element_type=jnp.float32)
        proj_ref[...] = p + bq_ref[...]

    s = lax.dot_general(proj_ref[...], keys_ref[...], (((1,), (1,)), ((), ())),
                        preferred_element_type=jnp.float32)
    s = s * jnp.float32(0.7 / math.sqrt(D)) + phi_ref[...]
    scores_ref[j] = s
    s3_ref[0] = s

    @pl.when(j == NKC - 1)
    def _():
        pieces = []
        for jj in range(NKC):
            sj = scores_ref[jj].reshape(RB1, SUB, CW)
            pieces.append(jnp.max(sj, axis=2))
        cm = jnp.concatenate(pieces, axis=1)
        lane = lax.broadcasted_iota(jnp.int32, (RB1, NCH), 1)
        riota = lax.broadcasted_iota(jnp.int32, (RB1, NCH), 0)[:, 0]
        growbase = (i * RB1 * 8)

        def body(k, c):
            m = jnp.max(c, axis=1)
            ck = jnp.min(jnp.where(c == m[:, None], lane, jnp.int32(NCH)),
                         axis=1)
            cid_ref[k] = ck
            gidx_ref[k] = ((ck >> 3) * jnp.int32(B * 8) + growbase
                           + riota * 8 + (ck & 7))
            return jnp.where(lane == ck[:, None], NEG, c)

        lax.fori_loop(0, NCAND, body, cm)


_score_cand = pl.pallas_call(
    _score_cand_body,
    grid=(NR1, NKC),
    in_specs=[
        pl.BlockSpec((RB1, D), lambda i, j: (i, 0)),
        pl.BlockSpec((D, D), lambda i, j: (0, 0)),
        pl.BlockSpec((1, D), lambda i, j: (0, 0)),
        pl.BlockSpec((KC, D), lambda i, j: (j, 0)),
        pl.BlockSpec((1, KC), lambda i, j: (0, j)),
    ],
    out_specs=[
        pl.BlockSpec((1, RB1, KC), lambda i, j: (j, i, 0)),
        pl.BlockSpec((NCAND, RB1), lambda i, j: (0, i)),
        pl.BlockSpec((NCAND, RB1), lambda i, j: (0, i)),
    ],
    out_shape=[
        jax.ShapeDtypeStruct((NKC, B, KC), jnp.float32),
        jax.ShapeDtypeStruct((NCAND, B), jnp.int32),
        jax.ShapeDtypeStruct((NCAND, B), jnp.int32),
    ],
    scratch_shapes=[
        pltpu.VMEM((RB1, D), jnp.float32),
        pltpu.VMEM((NKC, RB1, KC), jnp.float32),
    ],
    compiler_params=pltpu.CompilerParams(
        dimension_semantics=("parallel", "arbitrary")),
)


def _topk_body(c_in_ref, g_in_ref, idx_ref, w_ref, cand_ref, vals_ref):
    c0 = c_in_ref[...]
    cand_ref[...] = c0
    gcol = g_in_ref[...]
    m0 = jnp.max(c0, axis=1)

    def body(k, m):
        cand = cand_ref[...]
        idxk = jnp.min(jnp.where(cand == m[:, None], gcol, jnp.int32(BIG)),
                       axis=1)
        idx_ref[k] = idxk
        vals_ref[k] = m
        newc = jnp.where(gcol == idxk[:, None], NEG, cand)
        cand_ref[...] = newc
        return jnp.max(newc, axis=1)

    lax.fori_loop(0, K, body, m0)

    vals = vals_ref[...]
    mx = jnp.max(vals, axis=0)
    e = jnp.exp(vals - mx[None, :])
    w_ref[...] = e / jnp.sum(e, axis=0)[None, :]


_topk = pl.pallas_call(
    _topk_body,
    grid=(NR2,),
    in_specs=[
        pl.BlockSpec((RB2, CANDW), lambda i: (i, 0)),
        pl.BlockSpec((RB2, CANDW), lambda i: (i, 0)),
    ],
    out_specs=[
        pl.BlockSpec((K, RB2), lambda i: (0, i)),
        pl.BlockSpec((K, RB2), lambda i: (0, i)),
    ],
    out_shape=[
        jax.ShapeDtypeStruct((K, B), jnp.int32),
        jax.ShapeDtypeStruct((K, B), jnp.float32),
    ],
    scratch_shapes=[
        pltpu.VMEM((RB2, CANDW), jnp.float32),
        pltpu.VMEM((K, RB2), jnp.float32),
    ],
    compiler_params=pltpu.CompilerParams(
        dimension_semantics=("parallel",)),
)


SC_CORES, SC_SUBCORES = 2, 16
NW = SC_CORES * SC_SUBCORES
CH = 128


@functools.cache
def _make_gather_rows(nrows, width):
    per_w = nrows // NW
    nchunk = per_w // CH
    mesh = plsc.VectorSubcoreMesh(core_axis_name="c", subcore_axis_name="s")

    @functools.partial(
        pl.kernel,
        mesh=mesh,
        out_type=jax.ShapeDtypeStruct((nrows, width), jnp.float32),
        scratch_types=[
            pltpu.VMEM((CH,), jnp.int32),
            pltpu.VMEM((CH, width), jnp.float32),
            pltpu.SemaphoreType.DMA,
        ],
    )
    def _gather_rows(idx_hbm, table_hbm, out_hbm, idx_v, rows_v, sem):
        wid = lax.axis_index("s") * SC_CORES + lax.axis_index("c")
        base = wid * per_w
        for t in range(nchunk):
            off = base + t * CH
            pltpu.sync_copy(idx_hbm.at[pl.ds(off, CH)], idx_v)
            pltpu.async_copy(table_hbm.at[idx_v], rows_v, sem).wait()
            pltpu.sync_copy(rows_v, out_hbm.at[pl.ds(off, CH)])

    return _gather_rows


def kernel(query, Wq, bq, memory_keys, memory_values, phi_scores, top_k):
    del top_k
    phi_b = (phi_scores * jnp.float32(0.3)
             / (jnp.max(phi_scores) + jnp.float32(1e-8))).reshape(1, M)
    scores3, cid, gidx = _score_cand(query, Wq, bq.reshape(1, D),
                                     memory_keys, phi_b)
    table1 = scores3.reshape(NKC * B * SUB, CW)
    idx1 = gidx.T.reshape(B * NCAND)
    cands = _make_gather_rows(B * NCAND, CW)(idx1, table1).reshape(B, CANDW)
    gcol2 = (cid.T[:, :, None] * CW
             + jnp.arange(CW, dtype=jnp.int32)).reshape(B, CANDW)
    idx_t, w_t = _topk(cands, gcol2)
    idx = idx_t.T.reshape(B * K)
    retrieved = _make_gather_rows(B * K, D)(idx, memory_values)
    return retrieved.reshape(B, K, D), w_t.T
